# baseline (speedup 1.0000x reference)
import functools

import jax
import jax.numpy as jnp
from jax import lax
from jax.experimental import pallas as pl
from jax.experimental.pallas import tpu as pltpu
from jax.experimental.pallas import tpu_sc as plsc

_NC = 2
_NS = 16
_NW = _NC * _NS
_L = 16
_BLK = 25600
_TABLE_PAD = 1024


@functools.partial(jax.jit, static_argnums=(2,))
def _lookup_call(table, flat_data, n_total):
    per_w = n_total // _NW
    nblk = per_w // _BLK
    vecs = _BLK // _L
    mesh = plsc.VectorSubcoreMesh(core_axis_name="c", subcore_axis_name="s")

    @functools.partial(
        pl.kernel,
        mesh=mesh,
        out_type=jax.ShapeDtypeStruct((n_total,), jnp.int32),
        scratch_types=[
            pltpu.VMEM((_TABLE_PAD,), jnp.int32),
            pltpu.VMEM((2, _BLK), jnp.int32),
            pltpu.VMEM((2, _BLK), jnp.int32),
            pltpu.SemaphoreType.DMA,
            pltpu.SemaphoreType.DMA,
            pltpu.SemaphoreType.DMA,
            pltpu.SemaphoreType.DMA,
        ],
        compiler_params=pltpu.CompilerParams(needs_layout_passes=False),
    )
    def lookup(table_hbm, data_hbm, out_hbm, table_v, idx_v, res_v,
               in_sem0, in_sem1, out_sem0, out_sem1):
        wid = lax.axis_index("s") * _NC + lax.axis_index("c")
        base = wid * per_w
        in_sems = (in_sem0, in_sem1)
        out_sems = (out_sem0, out_sem1)
        pltpu.sync_copy(table_hbm, table_v)

        in_h = {}
        out_h = {}
        in_h[0] = pltpu.async_copy(
            data_hbm.at[pl.ds(base, _BLK)], idx_v.at[0], in_sems[0])
        for b in range(nblk):
            s = b % 2
            if b + 1 < nblk:
                in_h[b + 1] = pltpu.async_copy(
                    data_hbm.at[pl.ds(base + (b + 1) * _BLK, _BLK)],
                    idx_v.at[(b + 1) % 2], in_sems[(b + 1) % 2])
            in_h[b].wait()
            if b >= 2:
                out_h[b - 2].wait()

            @pl.loop(0, vecs, unroll=8)
            def vec_body(i):
                sl = pl.ds(i * _L, _L)
                res_v[s, sl] = plsc.load_gather(table_v, [idx_v[s, sl]])

            out_h[b] = pltpu.async_copy(
                res_v.at[s], out_hbm.at[pl.ds(base + b * _BLK, _BLK)],
                out_sems[s])
        for b in range(max(nblk - 2, 0), nblk):
            out_h[b].wait()

    return lookup(table, flat_data)


def kernel(data, table):
    n = data.shape[0] * data.shape[1]
    flat = data.reshape((n,))
    table_p = jnp.zeros((_TABLE_PAD,), jnp.int32).at[: table.shape[0]].set(table)
    out = _lookup_call(table_p, flat, n)
    return out.reshape(data.shape)


# speedup vs baseline: 2.0463x; 2.0463x over previous
import functools

import jax
import jax.numpy as jnp
from jax import lax
from jax.experimental import pallas as pl
from jax.experimental.pallas import tpu as pltpu
from jax.experimental.pallas import tpu_sc as plsc

_NC = 2
_NS = 16
_NW = _NC * _NS
_L = 16
_BR = 64
_TABLE_PAD = 1024


@functools.partial(jax.jit, static_argnums=(2, 3))
def _lookup_call(table, data, n_rows, n_cols):
    rows_per_w = n_rows // _NW
    nblk = rows_per_w // _BR
    col_starts = list(range(0, n_cols - _L + 1, _L))
    if col_starts[-1] + _L < n_cols:
        col_starts.append(n_cols - _L)
    mesh = plsc.VectorSubcoreMesh(core_axis_name="c", subcore_axis_name="s")

    @functools.partial(
        pl.kernel,
        mesh=mesh,
        out_type=jax.ShapeDtypeStruct((n_rows, n_cols), jnp.int32),
        scratch_types=[
            pltpu.VMEM((_TABLE_PAD,), jnp.int32),
            pltpu.VMEM((_BR, n_cols), jnp.int32),
            pltpu.VMEM((_BR, n_cols), jnp.int32),
            pltpu.VMEM((_BR, n_cols), jnp.int32),
            pltpu.VMEM((_BR, n_cols), jnp.int32),
            pltpu.SemaphoreType.DMA,
            pltpu.SemaphoreType.DMA,
            pltpu.SemaphoreType.DMA,
            pltpu.SemaphoreType.DMA,
        ],
        compiler_params=pltpu.CompilerParams(needs_layout_passes=False),
    )
    def lookup(table_hbm, data_hbm, out_hbm, table_v, idx_v0, idx_v1,
               res_v0, res_v1, in_sem0, in_sem1, out_sem0, out_sem1):
        wid = lax.axis_index("s") * _NC + lax.axis_index("c")
        base = wid * rows_per_w
        idx_bufs = (idx_v0, idx_v1)
        res_bufs = (res_v0, res_v1)
        in_sems = (in_sem0, in_sem1)
        out_sems = (out_sem0, out_sem1)
        pltpu.sync_copy(table_hbm, table_v)

        in_h = {}
        out_h = {}
        in_h[0] = pltpu.async_copy(
            data_hbm.at[pl.ds(base, _BR), :], idx_bufs[0], in_sems[0])
        for b in range(nblk):
            s = b % 2
            if b + 1 < nblk:
                in_h[b + 1] = pltpu.async_copy(
                    data_hbm.at[pl.ds(base + (b + 1) * _BR, _BR), :],
                    idx_bufs[(b + 1) % 2], in_sems[(b + 1) % 2])
            in_h[b].wait()
            if b >= 2:
                out_h[b - 2].wait()

            @pl.loop(0, _BR)
            def row_body(r):
                for c0 in col_starts:
                    sl = pl.ds(c0, _L)
                    res_bufs[s][r, sl] = plsc.load_gather(
                        table_v, [idx_bufs[s][r, sl]])

            out_h[b] = pltpu.async_copy(
                res_bufs[s], out_hbm.at[pl.ds(base + b * _BR, _BR), :],
                out_sems[s])
        for b in range(max(nblk - 2, 0), nblk):
            out_h[b].wait()

    return lookup(table, data)


def kernel(data, table):
    table_p = jnp.zeros((_TABLE_PAD,), jnp.int32).at[: table.shape[0]].set(table)
    return _lookup_call(table_p, data, data.shape[0], data.shape[1])


# speedup vs baseline: 2.5733x; 1.2575x over previous
import functools

import jax
import jax.numpy as jnp
from jax import lax
from jax.experimental import pallas as pl
from jax.experimental.pallas import tpu as pltpu
from jax.experimental.pallas import tpu_sc as plsc

_NC = 2
_NS = 16
_NW = _NC * _NS
_L = 16
_BR = 64
_TABLE_PAD = 1024


@functools.partial(jax.jit, static_argnums=(2, 3))
def _lookup_call(table, data, n_rows, n_cols):
    rows_per_w = n_rows // _NW
    nblk = rows_per_w // _BR
    col_starts = list(range(0, n_cols - _L + 1, _L))
    if col_starts[-1] + _L < n_cols:
        col_starts.append(n_cols - _L)
    mesh = plsc.VectorSubcoreMesh(core_axis_name="c", subcore_axis_name="s")

    @functools.partial(
        pl.kernel,
        mesh=mesh,
        out_type=jax.ShapeDtypeStruct((n_rows, n_cols), jnp.int32),
        scratch_types=[
            pltpu.VMEM((_TABLE_PAD,), jnp.int32),
            pltpu.VMEM((_BR, n_cols), jnp.int32),
            pltpu.VMEM((_BR, n_cols), jnp.int32),
            pltpu.VMEM((_BR, n_cols), jnp.int32),
            pltpu.VMEM((_BR, n_cols), jnp.int32),
            pltpu.SemaphoreType.DMA,
            pltpu.SemaphoreType.DMA,
            pltpu.SemaphoreType.DMA,
            pltpu.SemaphoreType.DMA,
        ],
        compiler_params=pltpu.CompilerParams(needs_layout_passes=False),
    )
    def lookup(table_hbm, data_hbm, out_hbm, table_v, idx_v0, idx_v1,
               res_v0, res_v1, in_sem0, in_sem1, out_sem0, out_sem1):
        wid = lax.axis_index("s") * _NC + lax.axis_index("c")
        base = wid * rows_per_w
        idx_bufs = (idx_v0, idx_v1)
        res_bufs = (res_v0, res_v1)
        in_sems = (in_sem0, in_sem1)
        out_sems = (out_sem0, out_sem1)
        pltpu.sync_copy(table_hbm, table_v)

        in_h = {}
        out_h = {}
        in_h[0] = pltpu.async_copy(
            data_hbm.at[pl.ds(base, _BR), :], idx_bufs[0], in_sems[0])
        for b in range(nblk):
            s = b % 2
            if b + 1 < nblk:
                in_h[b + 1] = pltpu.async_copy(
                    data_hbm.at[pl.ds(base + (b + 1) * _BR, _BR), :],
                    idx_bufs[(b + 1) % 2], in_sems[(b + 1) % 2])
            in_h[b].wait()
            if b >= 2:
                out_h[b - 2].wait()

            @pl.loop(0, _BR)
            def row_body(r):
                ivs = [idx_bufs[s][r, pl.ds(c0, _L)] for c0 in col_starts]
                tvs = [plsc.load_gather(table_v, [iv]) for iv in ivs]
                for c0, tv in zip(col_starts, tvs):
                    res_bufs[s][r, pl.ds(c0, _L)] = tv

            out_h[b] = pltpu.async_copy(
                res_bufs[s], out_hbm.at[pl.ds(base + b * _BR, _BR), :],
                out_sems[s])
        for b in range(max(nblk - 2, 0), nblk):
            out_h[b].wait()

    return lookup(table, data)


def kernel(data, table):
    table_p = jnp.zeros((_TABLE_PAD,), jnp.int32).at[: table.shape[0]].set(table)
    return _lookup_call(table_p, data, data.shape[0], data.shape[1])


# speedup vs baseline: 4.6780x; 1.8179x over previous
import functools

import jax
import jax.numpy as jnp
from jax import lax
from jax.experimental import pallas as pl
from jax.experimental.pallas import tpu as pltpu
from jax.experimental.pallas import tpu_sc as plsc

_NC = 2
_NS = 16
_NW = _NC * _NS
_L = 16
_RB = 40
_TABLE_PAD = 1024


@functools.partial(jax.jit, static_argnums=(2, 3))
def _lookup_call(table, data_t, n_rows, n_cols):
    cols_per_w = n_cols // _NW
    nblk = n_rows // _RB
    nslots = cols_per_w // _L
    mesh = plsc.VectorSubcoreMesh(core_axis_name="c", subcore_axis_name="s")

    @functools.partial(
        pl.kernel,
        mesh=mesh,
        out_type=jax.ShapeDtypeStruct((n_rows, n_cols), jnp.int32),
        scratch_types=[
            pltpu.VMEM((_TABLE_PAD,), jnp.int32),
            pltpu.VMEM((_RB, cols_per_w), jnp.int32),
            pltpu.VMEM((_RB, cols_per_w), jnp.int32),
            pltpu.VMEM((_RB, cols_per_w), jnp.int32),
            pltpu.VMEM((_RB, cols_per_w), jnp.int32),
            pltpu.SemaphoreType.DMA,
            pltpu.SemaphoreType.DMA,
            pltpu.SemaphoreType.DMA,
            pltpu.SemaphoreType.DMA,
        ],
        compiler_params=pltpu.CompilerParams(needs_layout_passes=False),
    )
    def lookup(table_hbm, data_hbm, out_hbm, table_v, idx_v0, idx_v1,
               res_v0, res_v1, in_sem0, in_sem1, out_sem0, out_sem1):
        wid = lax.axis_index("s") * _NC + lax.axis_index("c")
        col0 = wid * cols_per_w
        idx_bufs = (idx_v0, idx_v1)
        res_bufs = (res_v0, res_v1)
        in_sems = (in_sem0, in_sem1)
        out_sems = (out_sem0, out_sem1)
        pltpu.sync_copy(table_hbm, table_v)

        in_h = {}
        out_h = {}
        in_h[0] = pltpu.async_copy(
            data_hbm.at[pl.ds(0, _RB), pl.ds(col0, cols_per_w)],
            idx_bufs[0], in_sems[0])
        for b in range(nblk):
            s = b % 2
            if b + 1 < nblk:
                in_h[b + 1] = pltpu.async_copy(
                    data_hbm.at[pl.ds((b + 1) * _RB, _RB),
                                pl.ds(col0, cols_per_w)],
                    idx_bufs[(b + 1) % 2], in_sems[(b + 1) % 2])
            in_h[b].wait()
            if b >= 2:
                out_h[b - 2].wait()

            @pl.loop(0, _RB)
            def row_body(r):
                ivs = [idx_bufs[s][r, pl.ds(c * _L, _L)]
                       for c in range(nslots)]
                tvs = [plsc.load_gather(table_v, [iv]) for iv in ivs]
                for c, tv in enumerate(tvs):
                    res_bufs[s][r, pl.ds(c * _L, _L)] = tv

            out_h[b] = pltpu.async_copy(
                res_bufs[s],
                out_hbm.at[pl.ds(b * _RB, _RB), pl.ds(col0, cols_per_w)],
                out_sems[s])
        for b in range(max(nblk - 2, 0), nblk):
            out_h[b].wait()

    return lookup(table, data_t)


def kernel(data, table):
    table_p = jnp.zeros((_TABLE_PAD,), jnp.int32).at[: table.shape[0]].set(table)
    out_t = _lookup_call(table_p, data.T, data.shape[1], data.shape[0])
    return out_t.T


# speedup vs baseline: 4.7013x; 1.0050x over previous
import functools

import jax
import jax.numpy as jnp
from jax import lax
from jax.experimental import pallas as pl
from jax.experimental.pallas import tpu as pltpu
from jax.experimental.pallas import tpu_sc as plsc

_NC = 2
_NS = 16
_NW = _NC * _NS
_L = 16
_RB = 40


@functools.partial(jax.jit, static_argnums=(2, 3))
def _lookup_call(table, data_t, n_rows, n_cols):
    cols_per_w = n_cols // _NW
    nblk = n_rows // _RB
    nslots = cols_per_w // _L
    mesh = plsc.VectorSubcoreMesh(core_axis_name="c", subcore_axis_name="s")

    @functools.partial(
        pl.kernel,
        mesh=mesh,
        out_type=jax.ShapeDtypeStruct((n_rows, n_cols), jnp.int32),
        scratch_types=[
            pltpu.VMEM((1000,), jnp.int32),
            pltpu.VMEM((_RB, cols_per_w), jnp.int32),
            pltpu.VMEM((_RB, cols_per_w), jnp.int32),
            pltpu.VMEM((_RB, cols_per_w), jnp.int32),
            pltpu.VMEM((_RB, cols_per_w), jnp.int32),
            pltpu.SemaphoreType.DMA,
            pltpu.SemaphoreType.DMA,
            pltpu.SemaphoreType.DMA,
            pltpu.SemaphoreType.DMA,
        ],
        compiler_params=pltpu.CompilerParams(needs_layout_passes=False),
    )
    def lookup(table_hbm, data_hbm, out_hbm, table_v, idx_v0, idx_v1,
               res_v0, res_v1, in_sem0, in_sem1, out_sem0, out_sem1):
        wid = lax.axis_index("s") * _NC + lax.axis_index("c")
        col0 = wid * cols_per_w
        idx_bufs = (idx_v0, idx_v1)
        res_bufs = (res_v0, res_v1)
        in_sems = (in_sem0, in_sem1)
        out_sems = (out_sem0, out_sem1)
        pltpu.sync_copy(table_hbm, table_v)

        in_h = {}
        out_h = {}
        in_h[0] = pltpu.async_copy(
            data_hbm.at[pl.ds(0, _RB), pl.ds(col0, cols_per_w)],
            idx_bufs[0], in_sems[0])
        for b in range(nblk):
            s = b % 2
            if b + 1 < nblk:
                in_h[b + 1] = pltpu.async_copy(
                    data_hbm.at[pl.ds((b + 1) * _RB, _RB),
                                pl.ds(col0, cols_per_w)],
                    idx_bufs[(b + 1) % 2], in_sems[(b + 1) % 2])
            in_h[b].wait()
            if b >= 2:
                out_h[b - 2].wait()

            @pl.loop(0, _RB)
            def row_body(r):
                ivs = [idx_bufs[s][r, pl.ds(c * _L, _L)]
                       for c in range(nslots)]
                tvs = [plsc.load_gather(table_v, [iv]) for iv in ivs]
                for c, tv in enumerate(tvs):
                    res_bufs[s][r, pl.ds(c * _L, _L)] = tv

            out_h[b] = pltpu.async_copy(
                res_bufs[s],
                out_hbm.at[pl.ds(b * _RB, _RB), pl.ds(col0, cols_per_w)],
                out_sems[s])
        for b in range(max(nblk - 2, 0), nblk):
            out_h[b].wait()

    return lookup(table, data_t)


def kernel(data, table):
    out_t = _lookup_call(table, data.T, data.shape[1], data.shape[0])
    return out_t.T
